# Initial kernel scaffold; baseline (speedup 1.0000x reference)
import functools

import jax
import jax.numpy as jnp
from jax import lax
from jax.experimental import pallas as pl
from jax.experimental.pallas import tpu as pltpu
from jax.experimental.pallas import tpu_sc as plsc

NUM_ENT = 100000
BSZ = 4
L = 16
NP = 100352
SL16 = NP // 16
SL32 = NP // 32
NW = 32
C = 2000

_MESH = plsc.VectorSubcoreMesh(core_axis_name="c", subcore_axis_name="s")


def _step_body(first, e_hbm, sub_hbm, obj_hbm, dp_hbm, parts_out, enorm_out,
               tb0, tb1, tb2, tb3, ac0, ac1, ac2, ac3,
               b0, b1, subv, objv, pv, valsv, sem):
    tabs = [tb0, tb1, tb2, tb3]
    accs = [ac0, ac1, ac2, ac3]
    cid = lax.axis_index("c")
    sid = lax.axis_index("s")
    sl = pl.ds(sid * SL16, SL16)

    for b in range(BSZ):
        if first:
            pltpu.sync_copy(e_hbm.at[b, sl], b0)
        else:
            pltpu.sync_copy(e_hbm.at[0, b, sl], b0)
            pltpu.sync_copy(e_hbm.at[1, b, sl], b1)

            def nbody(i, _):
                ix = pl.ds(i * L, L)
                v = b0[ix] + b1[ix]
                b0[ix] = v / jnp.maximum(v, 1.0)
                return 0

            lax.fori_loop(0, SL16 // L, nbody, 0)
        pltpu.sync_copy(b0, tabs[b].at[sl])
        if not first:
            @pl.when(cid == 0)
            def _():
                pltpu.sync_copy(b0, enorm_out.at[b, sl])

    def zbody(i, _):
        b1[pl.ds(i * L, L)] = jnp.zeros((L,), jnp.float32)
        return 0

    lax.fori_loop(0, SL16 // L, zbody, 0)
    for b in range(BSZ):
        pltpu.sync_copy(b1, accs[b].at[sl])

    plsc.subcore_barrier()

    w = cid * 16 + sid
    per_w = sub_hbm.shape[0] // NW
    base = w * per_w
    nch = per_w // C

    def chunk(g, _):
        off = base + g * C
        pltpu.sync_copy(sub_hbm.at[pl.ds(off, C)], subv)
        pltpu.sync_copy(obj_hbm.at[pl.ds(off, C)], objv)
        for b in range(BSZ):
            pltpu.sync_copy(dp_hbm.at[b, pl.ds(off, C)], pv)
            pltpu.async_copy(tabs[b].at[subv], valsv, sem).wait()

            def mul(i, _):
                ix = pl.ds(i * L, L)
                valsv[ix] = valsv[ix] * pv[ix]
                return 0

            lax.fori_loop(0, C // L, mul, 0)
            pltpu.sync_copy(valsv, accs[b].at[objv], add=True)
        return 0

    lax.fori_loop(0, nch, chunk, 0)

    plsc.subcore_barrier()

    for b in range(BSZ):
        pltpu.sync_copy(accs[b].at[sl], parts_out.at[cid, b, sl])


def _make_step(first):
    out_type = [jax.ShapeDtypeStruct((2, BSZ, NP), jnp.float32)]
    if not first:
        out_type.append(jax.ShapeDtypeStruct((BSZ, NP), jnp.float32))
    scratch = (
        [pltpu.VMEM_SHARED((NP,), jnp.float32) for _ in range(2 * BSZ)]
        + [pltpu.VMEM((SL16,), jnp.float32), pltpu.VMEM((SL16,), jnp.float32),
           pltpu.VMEM((C,), jnp.int32), pltpu.VMEM((C,), jnp.int32),
           pltpu.VMEM((C,), jnp.float32), pltpu.VMEM((C,), jnp.float32),
           pltpu.SemaphoreType.DMA]
    )
    body = functools.partial(_step_body, first)
    if first:
        def body_first(e_hbm, sub_hbm, obj_hbm, dp_hbm, parts_out, *rest):
            return body(e_hbm, sub_hbm, obj_hbm, dp_hbm, parts_out, None, *rest)
        fn = body_first
    else:
        fn = body
    return pl.kernel(fn, out_type=tuple(out_type), mesh=_MESH,
                     scratch_types=tuple(scratch))


def _combine_body(parts_hbm, e1_hbm, es_hbm, ql_hbm, a0_hbm, a1_hbm, fl_hbm,
                  out_hbm, q0, q1, q2, q3, q4, s0, s1, s2):
    cid = lax.axis_index("c")
    sid = lax.axis_index("s")
    w = cid * 16 + sid
    sl = pl.ds(w * SL32, SL32)
    for b in range(BSZ):
        pltpu.sync_copy(parts_hbm.at[0, b, sl], q0)
        pltpu.sync_copy(parts_hbm.at[1, b, sl], q1)
        pltpu.sync_copy(e1_hbm.at[b, sl], q2)
        pltpu.sync_copy(es_hbm.at[b, sl], q3)
        pltpu.sync_copy(ql_hbm.at[b, sl], q4)
        pltpu.sync_copy(a0_hbm.at[b], s0)
        pltpu.sync_copy(a1_hbm.at[b], s1)
        pltpu.sync_copy(fl_hbm.at[b], s2)
        a0 = s0[...]
        a1 = s1[...]
        fl = s2[...]

        def cbody(i, _):
            ix = pl.ds(i * L, L)
            v = q0[ix] + q1[ix]
            e2 = v / jnp.maximum(v, 1.0)
            last = a0 * q2[ix] + a1 * e2
            last = (1.0 - fl * q3[ix]) * last
            sig = 1.0 / (1.0 + jnp.exp(-q4[ix]))
            q0[ix] = last * sig
            return 0

        lax.fori_loop(0, SL32 // L, cbody, 0)
        pltpu.sync_copy(q0, out_hbm.at[b, sl])


_combine = pl.kernel(
    _combine_body,
    out_type=jax.ShapeDtypeStruct((BSZ, NP), jnp.float32),
    mesh=_MESH,
    scratch_types=tuple(
        [pltpu.VMEM((SL32,), jnp.float32) for _ in range(5)]
        + [pltpu.VMEM((L,), jnp.float32) for _ in range(3)]
    ),
)

_step_first = _make_step(True)
_step_next = _make_step(False)


def kernel(e_s, pair, d_prob, hop_attn_logits, q_mask_logits):
    sub = pair[:, 0]
    obj = pair[:, 1]
    pad = NP - NUM_ENT
    es_p = jnp.pad(e_s, ((0, 0), (0, pad)))
    ql_p = jnp.pad(q_mask_logits, ((0, 0), (0, pad)))
    attn = jax.nn.softmax(hop_attn_logits, axis=1)
    a0 = jnp.broadcast_to(attn[:, 0:1], (BSZ, L))
    a1 = jnp.broadcast_to(attn[:, 1:2], (BSZ, L))
    fl = jnp.broadcast_to(
        (jnp.argmax(hop_attn_logits, axis=1) == 1).astype(jnp.float32)[:, None],
        (BSZ, L))
    parts0 = _step_first(es_p, sub, obj, d_prob[0])
    parts1, e1 = _step_next(parts0, sub, obj, d_prob[1])
    out = _combine(parts1, e1, es_p, ql_p, a0, a1, fl)
    return out[:, :NUM_ENT]


# speedup vs baseline: 26.1319x; 26.1319x over previous
import functools

import jax
import jax.numpy as jnp
from jax import lax
from jax.experimental import pallas as pl
from jax.experimental.pallas import tpu as pltpu
from jax.experimental.pallas import tpu_sc as plsc

NUM_ENT = 100000
BSZ = 4
L = 16
NP = 100352
SL16 = NP // 16
SL32 = NP // 32
NW = 32
C = 2000

_MESH = plsc.VectorSubcoreMesh(core_axis_name="c", subcore_axis_name="s")


def _step_body(first, e_hbm, sub_hbm, obj_hbm, dp_hbm, parts_out, enorm_out,
               tb0, tb1, tb2, tb3, ac0, ac1, ac2, ac3,
               b0, b1, subv, objv, pv, valsv, sem):
    tabs = [tb0, tb1, tb2, tb3]
    accs = [ac0, ac1, ac2, ac3]
    cid = lax.axis_index("c")
    sid = lax.axis_index("s")
    ent0 = sid * SL16
    sl = pl.ds(ent0, SL16)

    for b in range(BSZ):
        if first:
            pltpu.sync_copy(e_hbm.at[pl.ds(b * NP + ent0, SL16)], b0)
        else:
            pltpu.sync_copy(e_hbm.at[pl.ds(b * NP + ent0, SL16)], b0)
            pltpu.sync_copy(e_hbm.at[pl.ds((BSZ + b) * NP + ent0, SL16)], b1)

            def nbody(i, _):
                ix = pl.ds(i * L, L)
                v = b0[ix] + b1[ix]
                b0[ix] = v / jnp.maximum(v, 1.0)
                return 0

            lax.fori_loop(0, SL16 // L, nbody, 0)
        pltpu.sync_copy(b0, tabs[b].at[sl])
        if not first:
            @pl.when(cid == 0)
            def _():
                pltpu.sync_copy(b0, enorm_out.at[pl.ds(b * NP + ent0, SL16)])

    def zbody(i, _):
        b1[pl.ds(i * L, L)] = jnp.zeros((L,), jnp.float32)
        return 0

    lax.fori_loop(0, SL16 // L, zbody, 0)
    for b in range(BSZ):
        pltpu.sync_copy(b1, accs[b].at[sl])

    plsc.subcore_barrier()

    w = cid * 16 + sid
    per_w = sub_hbm.shape[0] // NW
    base = w * per_w
    nch = per_w // C
    E = sub_hbm.shape[0]

    def chunk(g, _):
        off = base + g * C
        pltpu.sync_copy(sub_hbm.at[pl.ds(off, C)], subv)
        pltpu.sync_copy(obj_hbm.at[pl.ds(off, C)], objv)
        for b in range(BSZ):
            pltpu.sync_copy(dp_hbm.at[pl.ds(b * E + off, C)], pv)
            pltpu.async_copy(tabs[b].at[subv], valsv, sem).wait()

            def mul(i, _):
                ix = pl.ds(i * L, L)
                valsv[ix] = valsv[ix] * pv[ix]
                return 0

            lax.fori_loop(0, C // L, mul, 0)
            pltpu.sync_copy(valsv, accs[b].at[objv], add=True)
        return 0

    lax.fori_loop(0, nch, chunk, 0)

    plsc.subcore_barrier()

    for b in range(BSZ):
        pltpu.sync_copy(accs[b].at[sl],
                        parts_out.at[pl.ds((cid * BSZ + b) * NP + ent0, SL16)])


def _make_step(first):
    parts_t = jax.ShapeDtypeStruct((2 * BSZ * NP,), jnp.float32)
    out_type = parts_t if first else (parts_t,
                                      jax.ShapeDtypeStruct((BSZ * NP,), jnp.float32))
    scratch = (
        [pltpu.VMEM_SHARED((NP,), jnp.float32) for _ in range(2 * BSZ)]
        + [pltpu.VMEM((SL16,), jnp.float32), pltpu.VMEM((SL16,), jnp.float32),
           pltpu.VMEM((C,), jnp.int32), pltpu.VMEM((C,), jnp.int32),
           pltpu.VMEM((C,), jnp.float32), pltpu.VMEM((C,), jnp.float32),
           pltpu.SemaphoreType.DMA]
    )
    body = functools.partial(_step_body, first)
    if first:
        def body_first(e_hbm, sub_hbm, obj_hbm, dp_hbm, parts_out, *rest):
            return body(e_hbm, sub_hbm, obj_hbm, dp_hbm, parts_out, None, *rest)
        fn = body_first
    else:
        fn = body
    return pl.kernel(fn, out_type=out_type, mesh=_MESH,
                     scratch_types=tuple(scratch))


def _combine_body(parts_hbm, e1_hbm, es_hbm, ql_hbm, sc_hbm,
                  out_hbm, q0, q1, q2, q3, q4, s0, s1, s2):
    cid = lax.axis_index("c")
    sid = lax.axis_index("s")
    w = cid * 16 + sid
    ent0 = w * SL32
    for b in range(BSZ):
        pltpu.sync_copy(parts_hbm.at[pl.ds(b * NP + ent0, SL32)], q0)
        pltpu.sync_copy(parts_hbm.at[pl.ds((BSZ + b) * NP + ent0, SL32)], q1)
        pltpu.sync_copy(e1_hbm.at[pl.ds(b * NP + ent0, SL32)], q2)
        pltpu.sync_copy(es_hbm.at[pl.ds(b * NP + ent0, SL32)], q3)
        pltpu.sync_copy(ql_hbm.at[pl.ds(b * NP + ent0, SL32)], q4)
        pltpu.sync_copy(sc_hbm.at[pl.ds(b * L, L)], s0)
        pltpu.sync_copy(sc_hbm.at[pl.ds((BSZ + b) * L, L)], s1)
        pltpu.sync_copy(sc_hbm.at[pl.ds((2 * BSZ + b) * L, L)], s2)
        a0 = s0[...]
        a1 = s1[...]
        fl = s2[...]

        def cbody(i, _):
            ix = pl.ds(i * L, L)
            v = q0[ix] + q1[ix]
            e2 = v / jnp.maximum(v, 1.0)
            last = a0 * q2[ix] + a1 * e2
            last = (1.0 - fl * q3[ix]) * last
            sig = 1.0 / (1.0 + jnp.exp(-q4[ix]))
            q0[ix] = last * sig
            return 0

        lax.fori_loop(0, SL32 // L, cbody, 0)
        pltpu.sync_copy(q0, out_hbm.at[pl.ds(b * NP + ent0, SL32)])


_combine = pl.kernel(
    _combine_body,
    out_type=jax.ShapeDtypeStruct((BSZ * NP,), jnp.float32),
    mesh=_MESH,
    scratch_types=tuple(
        [pltpu.VMEM((SL32,), jnp.float32) for _ in range(5)]
        + [pltpu.VMEM((L,), jnp.float32) for _ in range(3)]
    ),
)

_step_first = _make_step(True)
_step_next = _make_step(False)


def kernel(e_s, pair, d_prob, hop_attn_logits, q_mask_logits):
    sub = pair[:, 0]
    obj = pair[:, 1]
    pad = NP - NUM_ENT
    es_p = jnp.pad(e_s, ((0, 0), (0, pad))).reshape(-1)
    ql_p = jnp.pad(q_mask_logits, ((0, 0), (0, pad))).reshape(-1)
    attn = jax.nn.softmax(hop_attn_logits, axis=1)
    a0 = jnp.broadcast_to(attn[:, 0:1], (BSZ, L))
    a1 = jnp.broadcast_to(attn[:, 1:2], (BSZ, L))
    fl = jnp.broadcast_to(
        (jnp.argmax(hop_attn_logits, axis=1) == 1).astype(jnp.float32)[:, None],
        (BSZ, L))
    scal = jnp.concatenate([a0.reshape(-1), a1.reshape(-1), fl.reshape(-1)])
    parts0 = _step_first(es_p, sub, obj, d_prob[0].reshape(-1))
    parts1, e1 = _step_next(parts0, sub, obj, d_prob[1].reshape(-1))
    out = _combine(parts1, e1, es_p, ql_p, scal)
    return out.reshape(BSZ, NP)[:, :NUM_ENT]


# speedup vs baseline: 34.9114x; 1.3360x over previous
import functools

import jax
import jax.numpy as jnp
from jax import lax
from jax.experimental import pallas as pl
from jax.experimental.pallas import tpu as pltpu
from jax.experimental.pallas import tpu_sc as plsc

NUM_ENT = 100000
BSZ = 4
L = 16
NP = 100352
SL16 = NP // 16
SL32 = NP // 32
NW = 32
C = 2000

_MESH = plsc.VectorSubcoreMesh(core_axis_name="c", subcore_axis_name="s")


def _step_body(first, e_hbm, sub_hbm, obj_hbm, dp_hbm, parts_out, enorm_out,
               *sc):
    sc = list(sc)
    tabs = sc[0:4]
    accs = sc[4:8]
    b0, b1 = sc[8:10]
    subs = sc[10:12]
    objs = sc[12:14]
    ps = [sc[14:18], sc[18:22]]
    vals = [sc[22:26], sc[26:30]]
    sem_in, sem_g, sem_s = sc[30:33]
    cid = lax.axis_index("c")
    sid = lax.axis_index("s")
    ent0 = sid * SL16
    sl = pl.ds(ent0, SL16)

    for b in range(BSZ):
        if first:
            pltpu.sync_copy(e_hbm.at[pl.ds(b * NP + ent0, SL16)], b0)
        else:
            pltpu.sync_copy(e_hbm.at[pl.ds(b * NP + ent0, SL16)], b0)
            pltpu.sync_copy(e_hbm.at[pl.ds((BSZ + b) * NP + ent0, SL16)], b1)

            def nbody(i, _):
                ix = pl.ds(i * L, L)
                v = b0[ix] + b1[ix]
                b0[ix] = v / jnp.maximum(v, 1.0)
                return 0

            lax.fori_loop(0, SL16 // L, nbody, 0)
        pltpu.sync_copy(b0, tabs[b].at[sl])
        if not first:
            @pl.when(cid == 0)
            def _():
                pltpu.sync_copy(b0, enorm_out.at[pl.ds(b * NP + ent0, SL16)])

    def zbody(i, _):
        b1[pl.ds(i * L, L)] = jnp.zeros((L,), jnp.float32)
        return 0

    lax.fori_loop(0, SL16 // L, zbody, 0)
    for b in range(BSZ):
        pltpu.sync_copy(b1, accs[b].at[sl])

    plsc.subcore_barrier()

    w = cid * 16 + sid
    per_w = sub_hbm.shape[0] // NW
    base = w * per_w
    nch = per_w // C
    E = sub_hbm.shape[0]

    def linear_copies(g, slot):
        off = base + g * C
        yield sub_hbm.at[pl.ds(off, C)], subs[slot]
        yield obj_hbm.at[pl.ds(off, C)], objs[slot]
        for b in range(BSZ):
            yield dp_hbm.at[pl.ds(b * E + off, C)], ps[slot][b]

    def issue_linear(g, slot):
        for src, dst in linear_copies(g, slot):
            pltpu.async_copy(src, dst, sem_in)

    def wait_linear(g, slot):
        for src, dst in linear_copies(g, slot):
            pltpu.make_async_copy(src, dst, sem_in).wait()

    issue_linear(0, 0)

    def pair_of_chunks(gg):
        for half in range(2):
            g = gg + half
            slot = half
            wait_linear(g, slot)

            @pl.when(g + 1 < nch)
            def _():
                issue_linear(g + 1, 1 - slot)

            gds = [pltpu.async_copy(tabs[b].at[subs[slot]], vals[slot][b],
                                    sem_g) for b in range(BSZ)]
            sds = []
            for b in range(BSZ):
                gds[b].wait()
                vb = vals[slot][b]
                pb = ps[slot][b]

                @plsc.parallel_loop(0, C // L, 1, unroll=4)
                def _(i):
                    ix = pl.ds(i * L, L)
                    vb[ix] = vb[ix] * pb[ix]

                sds.append(pltpu.async_copy(vb, accs[b].at[objs[slot]],
                                            sem_s, add=True))
            for d in sds:
                d.wait()

    pl.loop(0, nch, step=2)(pair_of_chunks)

    plsc.subcore_barrier()

    for b in range(BSZ):
        pltpu.sync_copy(accs[b].at[sl],
                        parts_out.at[pl.ds((cid * BSZ + b) * NP + ent0, SL16)])


def _make_step(first):
    parts_t = jax.ShapeDtypeStruct((2 * BSZ * NP,), jnp.float32)
    out_type = parts_t if first else (parts_t,
                                      jax.ShapeDtypeStruct((BSZ * NP,), jnp.float32))
    scratch = (
        [pltpu.VMEM_SHARED((NP,), jnp.float32) for _ in range(2 * BSZ)]
        + [pltpu.VMEM((SL16,), jnp.float32), pltpu.VMEM((SL16,), jnp.float32)]
        + [pltpu.VMEM((C,), jnp.int32) for _ in range(4)]
        + [pltpu.VMEM((C,), jnp.float32) for _ in range(8)]
        + [pltpu.VMEM((C,), jnp.float32) for _ in range(8)]
        + [pltpu.SemaphoreType.DMA for _ in range(3)]
    )
    body = functools.partial(_step_body, first)
    if first:
        def body_first(e_hbm, sub_hbm, obj_hbm, dp_hbm, parts_out, *rest):
            return body(e_hbm, sub_hbm, obj_hbm, dp_hbm, parts_out, None, *rest)
        fn = body_first
    else:
        fn = body
    return pl.kernel(fn, out_type=out_type, mesh=_MESH,
                     scratch_types=tuple(scratch))


def _combine_body(parts_hbm, e1_hbm, es_hbm, ql_hbm, sc_hbm,
                  out_hbm, q0, q1, q2, q3, q4, s0, s1, s2):
    cid = lax.axis_index("c")
    sid = lax.axis_index("s")
    w = cid * 16 + sid
    ent0 = w * SL32
    for b in range(BSZ):
        pltpu.sync_copy(parts_hbm.at[pl.ds(b * NP + ent0, SL32)], q0)
        pltpu.sync_copy(parts_hbm.at[pl.ds((BSZ + b) * NP + ent0, SL32)], q1)
        pltpu.sync_copy(e1_hbm.at[pl.ds(b * NP + ent0, SL32)], q2)
        pltpu.sync_copy(es_hbm.at[pl.ds(b * NP + ent0, SL32)], q3)
        pltpu.sync_copy(ql_hbm.at[pl.ds(b * NP + ent0, SL32)], q4)
        pltpu.sync_copy(sc_hbm.at[pl.ds(b * L, L)], s0)
        pltpu.sync_copy(sc_hbm.at[pl.ds((BSZ + b) * L, L)], s1)
        pltpu.sync_copy(sc_hbm.at[pl.ds((2 * BSZ + b) * L, L)], s2)
        a0 = s0[...]
        a1 = s1[...]
        fl = s2[...]

        def cbody(i, _):
            ix = pl.ds(i * L, L)
            v = q0[ix] + q1[ix]
            e2 = v / jnp.maximum(v, 1.0)
            last = a0 * q2[ix] + a1 * e2
            last = (1.0 - fl * q3[ix]) * last
            sig = 1.0 / (1.0 + jnp.exp(-q4[ix]))
            q0[ix] = last * sig
            return 0

        lax.fori_loop(0, SL32 // L, cbody, 0)
        pltpu.sync_copy(q0, out_hbm.at[pl.ds(b * NP + ent0, SL32)])


_combine = pl.kernel(
    _combine_body,
    out_type=jax.ShapeDtypeStruct((BSZ * NP,), jnp.float32),
    mesh=_MESH,
    scratch_types=tuple(
        [pltpu.VMEM((SL32,), jnp.float32) for _ in range(5)]
        + [pltpu.VMEM((L,), jnp.float32) for _ in range(3)]
    ),
)

_step_first = _make_step(True)
_step_next = _make_step(False)


def kernel(e_s, pair, d_prob, hop_attn_logits, q_mask_logits):
    sub = pair[:, 0]
    obj = pair[:, 1]
    pad = NP - NUM_ENT
    es_p = jnp.pad(e_s, ((0, 0), (0, pad))).reshape(-1)
    ql_p = jnp.pad(q_mask_logits, ((0, 0), (0, pad))).reshape(-1)
    attn = jax.nn.softmax(hop_attn_logits, axis=1)
    a0 = jnp.broadcast_to(attn[:, 0:1], (BSZ, L))
    a1 = jnp.broadcast_to(attn[:, 1:2], (BSZ, L))
    fl = jnp.broadcast_to(
        (jnp.argmax(hop_attn_logits, axis=1) == 1).astype(jnp.float32)[:, None],
        (BSZ, L))
    scal = jnp.concatenate([a0.reshape(-1), a1.reshape(-1), fl.reshape(-1)])
    parts0 = _step_first(es_p, sub, obj, d_prob[0].reshape(-1))
    parts1, e1 = _step_next(parts0, sub, obj, d_prob[1].reshape(-1))
    out = _combine(parts1, e1, es_p, ql_p, scal)
    return out.reshape(BSZ, NP)[:, :NUM_ENT]
